# baseline (speedup 1.0000x reference)
import functools

import jax
import jax.numpy as jnp
from jax import lax
from jax.experimental import pallas as pl
from jax.experimental.pallas import tpu as pltpu
from jax.experimental.pallas import tpu_sc as plsc

NC = 2
NS = 16
NW = NC * NS
CP = 16


@functools.lru_cache(maxsize=None)
def _build(B, V, D):
    npairs = B // 2
    pairs_per_w = npairs // NW
    nchunk = pairs_per_w // CP
    vp = V // 2
    mesh = plsc.VectorSubcoreMesh(core_axis_name="c", subcore_axis_name="s")

    @functools.partial(
        pl.kernel,
        mesh=mesh,
        out_type=jax.ShapeDtypeStruct((npairs, D), jnp.int32),
        scratch_types=[
            pltpu.VMEM((2 * pairs_per_w + 16,), jnp.int32),
            pltpu.VMEM((nchunk, 2 * CP), jnp.int32),
            pltpu.VMEM((D,), jnp.int32),
            pltpu.VMEM((2 * CP, D), jnp.int32),
            pltpu.VMEM((2 * CP, D), jnp.int32),
            pltpu.VMEM((CP, D), jnp.int32),
            pltpu.VMEM((CP, D), jnp.int32),
            pltpu.SemaphoreType.DMA,
            pltpu.SemaphoreType.DMA,
            pltpu.SemaphoreType.DMA,
            pltpu.SemaphoreType.DMA,
        ],
    )
    def gather_kernel(ids_hbm, tpair_hbm, lastu_hbm, z_hbm,
                      ids_v, idx2, lastu_v, g0, g1, o0, o1,
                      gs0, gs1, ws0, ws1):
        wid = lax.axis_index("s") * NC + lax.axis_index("c")
        seqlen = ids_hbm.shape[1]
        per_w = 2 * pairs_per_w
        flat0 = wid * per_w
        pltpu.sync_copy(
            ids_hbm.at[flat0 // seqlen, pl.ds(flat0 % seqlen, per_w)],
            ids_v.at[pl.ds(0, per_w)])
        pltpu.sync_copy(lastu_hbm, lastu_v)

        def fill_idx(i, _):
            v = ids_v[pl.ds(i * 16, 16)]
            p = jnp.minimum(lax.shift_right_logical(v, 1), vp - 1)
            idx2[i // ((2 * CP) // 16), pl.ds((i % ((2 * CP) // 16)) * 16, 16)] = p
            return ()
        lax.fori_loop(0, per_w // 16, fill_idx, (), unroll=4)

        gbufs = (g0, g1)
        obufs = (o0, o1)
        gsems = (gs0, gs1)
        wsems = (ws0, ws1)
        hw = [None, None]

        hg0 = pltpu.async_copy(tpair_hbm.at[idx2.at[0]], g0, gs0)
        hg = [hg0, None]

        nvec = D // 16
        base_pair = wid * pairs_per_w

        for c in range(nchunk):
            b = c % 2
            if c + 1 < nchunk:
                nb = 1 - b
                hg[nb] = pltpu.async_copy(
                    tpair_hbm.at[idx2.at[c + 1]], gbufs[nb], gsems[nb])
            hg[b].wait()
            gbuf = gbufs[b]
            obuf = obufs[b]
            if hw[b] is not None:
                hw[b].wait()
                hw[b] = None

            def repack(m, _):
                idv = ids_v[pl.ds(c * 2 * CP + 2 * m, 16)]
                ida = idv[0]
                idb = idv[1]
                sa = (ida & 1) * 16
                sb = (1 - (idb & 1)) * 16

                def inner(j, _):
                    a = gbuf[2 * m, pl.ds(j * 16, 16)]
                    bb = gbuf[2 * m + 1, pl.ds(j * 16, 16)]
                    t1 = lax.shift_right_logical(a, sa) & 0xFFFF
                    t2 = lax.shift_left(bb, sb) & jnp.int32(-65536)
                    obuf[m, pl.ds(j * 16, 16)] = t1 | t2
                    return ()
                lax.fori_loop(0, nvec, inner, (), unroll=8)

                @pl.when(ida == V - 1)
                def _():
                    def fixa(j, _):
                        w = obuf[m, pl.ds(j * 16, 16)]
                        lo = lastu_v[pl.ds(j * 16, 16)]
                        obuf[m, pl.ds(j * 16, 16)] = (
                            (w & jnp.int32(-65536)) | lo)
                        return ()
                    lax.fori_loop(0, nvec, fixa, ())

                @pl.when(idb == V - 1)
                def _():
                    def fixb(j, _):
                        w = obuf[m, pl.ds(j * 16, 16)]
                        hi = lax.shift_left(lastu_v[pl.ds(j * 16, 16)], 16)
                        obuf[m, pl.ds(j * 16, 16)] = (w & 0xFFFF) | hi
                        return ()
                    lax.fori_loop(0, nvec, fixb, ())
                return ()

            lax.fori_loop(0, CP, repack, ())
            hw[b] = pltpu.async_copy(
                obuf, z_hbm.at[pl.ds(base_pair + c * CP, CP)], wsems[b])
        for b in range(2):
            if hw[b] is not None:
                hw[b].wait()

    return gather_kernel


def kernel(input_ids, table):
    batch, seqlen = input_ids.shape
    vocab, dim = table.shape
    B = batch * seqlen
    vp = vocab // 2
    tpair = lax.bitcast_convert_type(
        jnp.stack([table[0:2 * vp:2], table[1:2 * vp:2]], axis=-1), jnp.int32)
    lastu = lax.bitcast_convert_type(table[vocab - 1], jnp.uint16).astype(
        jnp.int32)
    z = _build(B, vocab, dim)(input_ids.astype(jnp.int32), tpair, lastu)
    out = lax.bitcast_convert_type(z, jnp.bfloat16)
    return out.transpose(0, 2, 1).reshape(batch, seqlen, dim)


# speedup vs baseline: 1.8581x; 1.8581x over previous
import functools

import jax
import jax.numpy as jnp
from jax import lax
from jax.experimental import pallas as pl
from jax.experimental.pallas import tpu as pltpu
from jax.experimental.pallas import tpu_sc as plsc

NC = 2
NS = 16
NW = NC * NS
CP = 16
BLK = 144


def _detile_body(x_ref, o_ref):
    o_ref[...] = pltpu.bitcast(x_ref[...], jnp.int32)


@functools.lru_cache(maxsize=None)
def _detile(vocab, dim):
    grid = (vocab + BLK - 1) // BLK
    return pl.pallas_call(
        _detile_body,
        grid=(grid,),
        in_specs=[pl.BlockSpec((BLK, dim), lambda i: (i, 0))],
        out_specs=pl.BlockSpec((BLK // 2, dim), lambda i: (i, 0)),
        out_shape=jax.ShapeDtypeStruct((grid * BLK // 2, dim), jnp.int32),
    )


@functools.lru_cache(maxsize=None)
def _build(B, VP, D):
    npairs = B // 2
    pairs_per_w = npairs // NW
    nchunk = pairs_per_w // CP
    mesh = plsc.VectorSubcoreMesh(core_axis_name="c", subcore_axis_name="s")

    @functools.partial(
        pl.kernel,
        mesh=mesh,
        out_type=jax.ShapeDtypeStruct((npairs, D), jnp.int32),
        scratch_types=[
            pltpu.VMEM((2 * pairs_per_w + 16,), jnp.int32),
            pltpu.VMEM((nchunk, 2 * CP), jnp.int32),
            pltpu.VMEM((2 * CP, D), jnp.int32),
            pltpu.VMEM((2 * CP, D), jnp.int32),
            pltpu.VMEM((CP, D), jnp.int32),
            pltpu.VMEM((CP, D), jnp.int32),
            pltpu.SemaphoreType.DMA,
            pltpu.SemaphoreType.DMA,
            pltpu.SemaphoreType.DMA,
            pltpu.SemaphoreType.DMA,
        ],
    )
    def gather_kernel(ids_hbm, tpair_hbm, z_hbm,
                      ids_v, idx2, g0, g1, o0, o1,
                      gs0, gs1, ws0, ws1):
        wid = lax.axis_index("s") * NC + lax.axis_index("c")
        seqlen = ids_hbm.shape[1]
        per_w = 2 * pairs_per_w
        flat0 = wid * per_w
        pltpu.sync_copy(
            ids_hbm.at[flat0 // seqlen, pl.ds(flat0 % seqlen, per_w)],
            ids_v.at[pl.ds(0, per_w)])

        def fill_idx(i, _):
            v = ids_v[pl.ds(i * 16, 16)]
            p = lax.shift_right_logical(v, 1)
            idx2[i // ((2 * CP) // 16), pl.ds((i % ((2 * CP) // 16)) * 16, 16)] = p
            return ()
        lax.fori_loop(0, per_w // 16, fill_idx, (), unroll=4)

        gbufs = (g0, g1)
        obufs = (o0, o1)
        gsems = (gs0, gs1)
        wsems = (ws0, ws1)
        hw = [None, None]

        hg0 = pltpu.async_copy(tpair_hbm.at[idx2.at[0]], g0, gs0)
        hg = [hg0, None]

        nvec = D // 16
        half = nvec // 2
        base_pair = wid * pairs_per_w

        for c in range(nchunk):
            b = c % 2
            if c + 1 < nchunk:
                nb = 1 - b
                hg[nb] = pltpu.async_copy(
                    tpair_hbm.at[idx2.at[c + 1]], gbufs[nb], gsems[nb])
            hg[b].wait()
            gbuf = gbufs[b]
            obuf = obufs[b]
            if hw[b] is not None:
                hw[b].wait()
                hw[b] = None

            def repack(m, _):
                idv = ids_v[pl.ds(c * 2 * CP + 2 * m, 16)]
                ida = idv[0]
                idb = idv[1]
                sa = (ida & 1) * 16
                sb = (1 - (idb & 1)) * 16

                def inner(j, _):
                    a0 = gbuf[2 * m, pl.ds(j * 16, 16)]
                    b0 = gbuf[2 * m + 1, pl.ds(j * 16, 16)]
                    a1 = gbuf[2 * m, pl.ds((j + half) * 16, 16)]
                    b1 = gbuf[2 * m + 1, pl.ds((j + half) * 16, 16)]
                    z0 = (lax.shift_right_logical(a0, sa) & 0xFFFF) | (
                        lax.shift_left(b0, sb) & jnp.int32(-65536))
                    z1 = (lax.shift_right_logical(a1, sa) & 0xFFFF) | (
                        lax.shift_left(b1, sb) & jnp.int32(-65536))
                    obuf[m, pl.ds(j * 16, 16)] = z0
                    obuf[m, pl.ds((j + half) * 16, 16)] = z1
                    return ()
                lax.fori_loop(0, half, inner, (), unroll=4)
                return ()

            lax.fori_loop(0, CP, repack, ())
            hw[b] = pltpu.async_copy(
                obuf, z_hbm.at[pl.ds(base_pair + c * CP, CP)], wsems[b])
        for b in range(2):
            if hw[b] is not None:
                hw[b].wait()

    return gather_kernel


def kernel(input_ids, table):
    batch, seqlen = input_ids.shape
    vocab, dim = table.shape
    B = batch * seqlen
    tpair = _detile(vocab, dim)(table)
    z = _build(B, tpair.shape[0], dim)(input_ids.astype(jnp.int32), tpair)
    out = lax.bitcast_convert_type(z, jnp.bfloat16)
    return out.transpose(0, 2, 1).reshape(batch, seqlen, dim)


# speedup vs baseline: 2.3134x; 1.2450x over previous
import functools

import jax
import jax.numpy as jnp
from jax import lax
from jax.experimental import pallas as pl
from jax.experimental.pallas import tpu as pltpu
from jax.experimental.pallas import tpu_sc as plsc

NC = 2
NS = 16
NW = NC * NS
CP = 16
BLK = 1024


def _detile_body(x_ref, o_ref):
    o_ref[...] = pltpu.bitcast(x_ref[...], jnp.int32)


@functools.lru_cache(maxsize=None)
def _detile(vocab, dim):
    grid = (vocab + BLK - 1) // BLK
    return pl.pallas_call(
        _detile_body,
        grid=(grid,),
        in_specs=[pl.BlockSpec((BLK, dim), lambda i: (i, 0))],
        out_specs=pl.BlockSpec((BLK // 2, dim), lambda i: (i, 0)),
        out_shape=jax.ShapeDtypeStruct((grid * BLK // 2, dim), jnp.int32),
    )


@functools.lru_cache(maxsize=None)
def _build(B, VP, D):
    npairs = B // 2
    pairs_per_w = npairs // NW
    nchunk = pairs_per_w // CP
    mesh = plsc.VectorSubcoreMesh(core_axis_name="c", subcore_axis_name="s")

    @functools.partial(
        pl.kernel,
        mesh=mesh,
        out_type=jax.ShapeDtypeStruct((npairs, D), jnp.int32),
        scratch_types=[
            pltpu.VMEM((2 * pairs_per_w + 16,), jnp.int32),
            pltpu.VMEM((nchunk, 2 * CP), jnp.int32),
            pltpu.VMEM((2 * CP, D), jnp.int32),
            pltpu.VMEM((2 * CP, D), jnp.int32),
            pltpu.VMEM((CP, D), jnp.int32),
            pltpu.VMEM((CP, D), jnp.int32),
            pltpu.SemaphoreType.DMA,
            pltpu.SemaphoreType.DMA,
            pltpu.SemaphoreType.DMA,
            pltpu.SemaphoreType.DMA,
        ],
    )
    def gather_kernel(ids_hbm, tpair_hbm, z_hbm,
                      ids_v, idx2, g0, g1, o0, o1,
                      gs0, gs1, ws0, ws1):
        wid = lax.axis_index("s") * NC + lax.axis_index("c")
        seqlen = ids_hbm.shape[1]
        per_w = 2 * pairs_per_w
        flat0 = wid * per_w
        pltpu.sync_copy(
            ids_hbm.at[flat0 // seqlen, pl.ds(flat0 % seqlen, per_w)],
            ids_v.at[pl.ds(0, per_w)])

        def fill_idx(i, _):
            v = ids_v[pl.ds(i * 16, 16)]
            p = lax.shift_right_logical(v, 1)
            idx2[i // ((2 * CP) // 16), pl.ds((i % ((2 * CP) // 16)) * 16, 16)] = p
            return ()
        lax.fori_loop(0, per_w // 16, fill_idx, (), unroll=4)

        gbufs = (g0, g1)
        obufs = (o0, o1)
        gsems = (gs0, gs1)
        wsems = (ws0, ws1)
        hw = [None, None]

        hg0 = pltpu.async_copy(tpair_hbm.at[idx2.at[0]], g0, gs0)
        hg = [hg0, None]

        nvec = D // 16
        half = nvec // 2
        base_pair = wid * pairs_per_w

        for c in range(nchunk):
            b = c % 2
            if c + 1 < nchunk:
                nb = 1 - b
                hg[nb] = pltpu.async_copy(
                    tpair_hbm.at[idx2.at[c + 1]], gbufs[nb], gsems[nb])
            hg[b].wait()
            gbuf = gbufs[b]
            obuf = obufs[b]
            if hw[b] is not None:
                hw[b].wait()
                hw[b] = None

            def repack(m, _):
                idv = ids_v[pl.ds(c * 2 * CP + 2 * m, 16)]
                ida = idv[0]
                idb = idv[1]
                sa = (ida & 1) * 16
                sb = (1 - (idb & 1)) * 16

                def inner(j, _):
                    a0 = gbuf[2 * m, pl.ds(j * 16, 16)]
                    b0 = gbuf[2 * m + 1, pl.ds(j * 16, 16)]
                    a1 = gbuf[2 * m, pl.ds((j + half) * 16, 16)]
                    b1 = gbuf[2 * m + 1, pl.ds((j + half) * 16, 16)]
                    z0 = (lax.shift_right_logical(a0, sa) & 0xFFFF) | (
                        lax.shift_left(b0, sb) & jnp.int32(-65536))
                    z1 = (lax.shift_right_logical(a1, sa) & 0xFFFF) | (
                        lax.shift_left(b1, sb) & jnp.int32(-65536))
                    obuf[m, pl.ds(j * 16, 16)] = z0
                    obuf[m, pl.ds((j + half) * 16, 16)] = z1
                    return ()
                lax.fori_loop(0, half, inner, (), unroll=4)
                return ()

            lax.fori_loop(0, CP, repack, ())
            hw[b] = pltpu.async_copy(
                obuf, z_hbm.at[pl.ds(base_pair + c * CP, CP)], wsems[b])
        for b in range(2):
            if hw[b] is not None:
                hw[b].wait()

    return gather_kernel


def kernel(input_ids, table):
    batch, seqlen = input_ids.shape
    vocab, dim = table.shape
    B = batch * seqlen
    tpair = _detile(vocab, dim)(table)
    z = _build(B, tpair.shape[0], dim)(input_ids.astype(jnp.int32), tpair)
    out = lax.bitcast_convert_type(z, jnp.bfloat16)
    return out.transpose(0, 2, 1).reshape(batch, seqlen, dim)


# speedup vs baseline: 2.3578x; 1.0192x over previous
import functools

import jax
import jax.numpy as jnp
from jax import lax
from jax.experimental import pallas as pl
from jax.experimental.pallas import tpu as pltpu
from jax.experimental.pallas import tpu_sc as plsc

NC = 2
NS = 16
NW = NC * NS
CP = 16
BLK = 2048


def _detile_body(x_ref, o_ref):
    o_ref[...] = pltpu.bitcast(x_ref[...], jnp.int32)


@functools.lru_cache(maxsize=None)
def _detile(vocab, dim):
    grid = (vocab + BLK - 1) // BLK
    return pl.pallas_call(
        _detile_body,
        grid=(grid,),
        in_specs=[pl.BlockSpec((BLK, dim), lambda i: (i, 0))],
        out_specs=pl.BlockSpec((BLK // 2, dim), lambda i: (i, 0)),
        out_shape=jax.ShapeDtypeStruct((grid * BLK // 2, dim), jnp.int32),
    )


@functools.lru_cache(maxsize=None)
def _build(B, VP, D):
    npairs = B // 2
    pairs_per_w = npairs // NW
    nchunk = pairs_per_w // CP
    mesh = plsc.VectorSubcoreMesh(core_axis_name="c", subcore_axis_name="s")

    @functools.partial(
        pl.kernel,
        mesh=mesh,
        out_type=jax.ShapeDtypeStruct((npairs, D), jnp.int32),
        scratch_types=[
            pltpu.VMEM((2 * pairs_per_w + 16,), jnp.int32),
            pltpu.VMEM((nchunk, 2 * CP), jnp.int32),
            pltpu.VMEM((2 * CP, D), jnp.int32),
            pltpu.VMEM((2 * CP, D), jnp.int32),
            pltpu.VMEM((CP, D), jnp.int32),
            pltpu.VMEM((CP, D), jnp.int32),
            pltpu.SemaphoreType.DMA,
            pltpu.SemaphoreType.DMA,
            pltpu.SemaphoreType.DMA,
            pltpu.SemaphoreType.DMA,
        ],
    )
    def gather_kernel(ids_hbm, tpair_hbm, z_hbm,
                      ids_v, idx2, g0, g1, o0, o1,
                      gs0, gs1, ws0, ws1):
        wid = lax.axis_index("s") * NC + lax.axis_index("c")
        seqlen = ids_hbm.shape[1]
        per_w = 2 * pairs_per_w
        flat0 = wid * per_w
        pltpu.sync_copy(
            ids_hbm.at[flat0 // seqlen, pl.ds(flat0 % seqlen, per_w)],
            ids_v.at[pl.ds(0, per_w)])

        def fill_idx(i, _):
            v = ids_v[pl.ds(i * 16, 16)]
            p = lax.shift_right_logical(v, 1)
            idx2[i // ((2 * CP) // 16), pl.ds((i % ((2 * CP) // 16)) * 16, 16)] = p
            return ()
        lax.fori_loop(0, per_w // 16, fill_idx, (), unroll=4)

        gbufs = (g0, g1)
        obufs = (o0, o1)
        gsems = (gs0, gs1)
        wsems = (ws0, ws1)
        hw = [None, None]

        hg0 = pltpu.async_copy(tpair_hbm.at[idx2.at[0]], g0, gs0)
        hg = [hg0, None]

        nvec = D // 16
        half = nvec // 2
        base_pair = wid * pairs_per_w

        for c in range(nchunk):
            b = c % 2
            if c + 1 < nchunk:
                nb = 1 - b
                hg[nb] = pltpu.async_copy(
                    tpair_hbm.at[idx2.at[c + 1]], gbufs[nb], gsems[nb])
            hg[b].wait()
            gbuf = gbufs[b]
            obuf = obufs[b]
            if hw[b] is not None:
                hw[b].wait()
                hw[b] = None

            def repack(m, _):
                idv = ids_v[pl.ds(c * 2 * CP + 2 * m, 16)]
                ida = idv[0]
                idb = idv[1]
                sa = (ida & 1) * 16
                sb = (1 - (idb & 1)) * 16

                def inner(j, _):
                    a0 = gbuf[2 * m, pl.ds(j * 16, 16)]
                    b0 = gbuf[2 * m + 1, pl.ds(j * 16, 16)]
                    a1 = gbuf[2 * m, pl.ds((j + half) * 16, 16)]
                    b1 = gbuf[2 * m + 1, pl.ds((j + half) * 16, 16)]
                    z0 = (lax.shift_right_logical(a0, sa) & 0xFFFF) | (
                        lax.shift_left(b0, sb) & jnp.int32(-65536))
                    z1 = (lax.shift_right_logical(a1, sa) & 0xFFFF) | (
                        lax.shift_left(b1, sb) & jnp.int32(-65536))
                    obuf[m, pl.ds(j * 16, 16)] = z0
                    obuf[m, pl.ds((j + half) * 16, 16)] = z1
                    return ()
                lax.fori_loop(0, half, inner, (), unroll=4)
                return ()

            lax.fori_loop(0, CP, repack, ())
            hw[b] = pltpu.async_copy(
                obuf, z_hbm.at[pl.ds(base_pair + c * CP, CP)], wsems[b])
        for b in range(2):
            if hw[b] is not None:
                hw[b].wait()

    return gather_kernel


def kernel(input_ids, table):
    batch, seqlen = input_ids.shape
    vocab, dim = table.shape
    B = batch * seqlen
    tpair = _detile(vocab, dim)(table)
    z = _build(B, tpair.shape[0], dim)(input_ids.astype(jnp.int32), tpair)
    out = lax.bitcast_convert_type(z, jnp.bfloat16)
    return out.transpose(0, 2, 1).reshape(batch, seqlen, dim)
